# baseline (speedup 1.0000x reference)
import jax
import jax.numpy as jnp
from jax import lax
from jax.experimental import pallas as pl
from jax.experimental.pallas import tpu as pltpu
from jax.experimental.pallas import tpu_sc as plsc

N = 10000
D = 128
DE = 16
E = 320000

NC = 2
NS = 16
NW = NC * NS
EW = E // NW
C = 80
NCHUNK = EW // C
NGRP = C // 16
NB = 2
ROWS_PER_TILE = N // NS
INV_SQRT_D = 1.0 / float(D) ** 0.5


def _ln_body(lam_ref, x_ref, e_ref, y_ref, e2_ref):
    x = x_ref[...]
    mean = jnp.mean(x, axis=-1, keepdims=True)
    var = jnp.mean((x - mean) ** 2, axis=-1, keepdims=True)
    y_ref[...] = (x - mean) / jnp.sqrt(var + 1e-5)
    scale = jnp.exp(0.5 * lam_ref[0])
    e2_ref[...] = e_ref[...] * scale


def _layernorm_and_scale(x, eigs, lambda0):
    return pl.pallas_call(
        _ln_body,
        out_shape=(
            jax.ShapeDtypeStruct((N, D), jnp.float32),
            jax.ShapeDtypeStruct((N, DE), jnp.float32),
        ),
        grid=(10,),
        in_specs=[
            pl.BlockSpec(memory_space=pltpu.SMEM),
            pl.BlockSpec((N // 10, D), lambda i: (i, 0)),
            pl.BlockSpec((N // 10, DE), lambda i: (i, 0)),
        ],
        out_specs=(
            pl.BlockSpec((N // 10, D), lambda i: (i, 0)),
            pl.BlockSpec((N // 10, DE), lambda i: (i, 0)),
        ),
    )(lambda0, x, eigs)


def _edge_body(y_hbm, eig_hbm, src_hbm, dst_hbm, acc_out, den_out,
               src_idx, dst_idx, ysrc, ydst, esrc, edst, valbuf, mbuf, zd,
               acc_sp, den_sp, sem_g, sem_s):
    cid = lax.axis_index("c")
    sid = lax.axis_index("s")
    wid = cid * NS + sid

    def _zero_rows(r, _):
        for j in range(D // 16):
            ysrc[0][r, pl.ds(j * 16, 16)] = jnp.zeros((16,), jnp.float32)
        return 0

    lax.fori_loop(0, C, _zero_rows, 0)

    def _zero_zd(i, _):
        zd[pl.ds(i * 16, 16)] = jnp.zeros((16,), jnp.float32)
        return 0

    lax.fori_loop(0, 800 // 16, _zero_zd, 0)

    for k in range(7):
        pltpu.sync_copy(ysrc[0],
                        acc_sp.at[pl.ds(sid * ROWS_PER_TILE + k * C, C)])
    pltpu.sync_copy(ysrc[0].at[pl.ds(0, 65)],
                    acc_sp.at[pl.ds(sid * ROWS_PER_TILE + 7 * C, 65)])
    @pl.when(sid < 12)
    def _():
        pltpu.sync_copy(zd, den_sp.at[pl.ds(sid * 800, 800)])

    @pl.when(sid == 12)
    def _():
        pltpu.sync_copy(zd.at[pl.ds(0, 400)], den_sp.at[pl.ds(9600, 400)])

    plsc.subcore_barrier()

    def _copy_idx(ck, b):
        pltpu.sync_copy(src_hbm.at[wid, ck], src_idx[b])
        pltpu.sync_copy(dst_hbm.at[wid, ck], dst_idx[b])

    def _issue_gather(b):
        pltpu.async_copy(y_hbm.at[src_idx[b]], ysrc[b], sem_g[b])
        pltpu.async_copy(y_hbm.at[dst_idx[b]], ydst[b], sem_g[b])
        pltpu.async_copy(eig_hbm.at[src_idx[b]], esrc[b], sem_g[b])
        pltpu.async_copy(eig_hbm.at[dst_idx[b]], edst[b], sem_g[b])

    def _wait_gather(b):
        pltpu.make_async_copy(y_hbm.at[src_idx[b]], ysrc[b], sem_g[b]).wait()
        pltpu.make_async_copy(y_hbm.at[dst_idx[b]], ydst[b], sem_g[b]).wait()
        pltpu.make_async_copy(eig_hbm.at[src_idx[b]], esrc[b],
                              sem_g[b]).wait()
        pltpu.make_async_copy(eig_hbm.at[dst_idx[b]], edst[b],
                              sem_g[b]).wait()

    def _issue_scatter(b):
        pltpu.async_copy(ysrc[b], acc_sp.at[src_idx[b]], sem_s[b], add=True)
        pltpu.async_copy(valbuf[b], den_sp.at[src_idx[b]], sem_s[b],
                         add=True)

    def _wait_scatter(b):
        pltpu.make_async_copy(ysrc[b], acc_sp.at[src_idx[b]],
                              sem_s[b]).wait()
        pltpu.make_async_copy(valbuf[b], den_sp.at[src_idx[b]],
                              sem_s[b]).wait()

    def _compute(b):
        lanes16 = lax.iota(jnp.int32, 16) * 16
        inv = jnp.full((16,), INV_SQRT_D, jnp.float32)

        def _group(g, _):
            for e in range(16):
                r = g * 16 + e
                qk = [ysrc[b][r, pl.ds(j * 16, 16)] for j in range(D // 16)]
                kk = [ydst[b][r, pl.ds(j * 16, 16)] for j in range(D // 16)]
                t0 = qk[0] * kk[0] + qk[1] * kk[1]
                t1 = qk[2] * kk[2] + qk[3] * kk[3]
                t2 = qk[4] * kk[4] + qk[5] * kk[5]
                t3 = qk[6] * kk[6] + qk[7] * kk[7]
                accy = (t0 + t1) + (t2 + t3)
                acc_t = accy * inv + esrc[b][r, :] * edst[b][r, :]
                mbuf[pl.ds(e * 16, 16)] = acc_t
            sv = plsc.load_gather(mbuf, [lanes16])
            for l in range(1, 16):
                sv = sv + plsc.load_gather(mbuf, [lanes16 + l])
            vals = jnp.minimum(jnp.exp(sv), 5.0)
            valbuf[b][pl.ds(g * 16, 16)] = vals
            for e in range(16):
                r = g * 16 + e
                val = plsc.load_gather(valbuf[b], [jnp.full((16,), r,
                                                           jnp.int32)])
                for j in range(D // 16):
                    ysrc[b][r, pl.ds(j * 16, 16)] = (
                        ydst[b][r, pl.ds(j * 16, 16)] * val)
            return 0

        lax.fori_loop(0, NGRP, _group, 0)

    _copy_idx(0, 0)
    _issue_gather(0)
    _copy_idx(1, 1)
    _issue_gather(1)
    _wait_gather(0)
    _compute(0)
    _issue_scatter(0)

    def _steady(m, _):
        for bo in range(NB):
            ck = 1 + 2 * m + bo
            b = (1 + bo) % NB
            nxt = (b + 1) % NB
            _wait_scatter(nxt)

            @pl.when(ck + 1 < NCHUNK)
            def _():
                _copy_idx(ck + 1, nxt)
                _issue_gather(nxt)

            _wait_gather(b)
            _compute(b)
            _issue_scatter(b)
        return 0

    lax.fori_loop(0, (NCHUNK - 1) // NB, _steady, 0)

    _wait_scatter(0)

    plsc.subcore_barrier()

    @pl.when(sid < 15)
    def _():
        pltpu.sync_copy(acc_sp.at[pl.ds(sid * 632, 632)],
                        acc_out.at[cid, pl.ds(sid * 632, 632)])

    @pl.when(sid == 15)
    def _():
        pltpu.sync_copy(acc_sp.at[pl.ds(9480, 520)],
                        acc_out.at[cid, pl.ds(9480, 520)])

    @pl.when(sid < 12)
    def _():
        pltpu.sync_copy(den_sp.at[pl.ds(sid * 800, 800)],
                        den_out.at[pl.ds(cid * N + sid * 800, 800)])

    @pl.when(sid == 12)
    def _():
        pltpu.sync_copy(den_sp.at[pl.ds(9600, 400)],
                        den_out.at[pl.ds(cid * N + 9600, 400)])


def _edge_pass(y, eigs2, src3, dst3):
    mesh = plsc.VectorSubcoreMesh(core_axis_name="c", subcore_axis_name="s")
    f = pl.kernel(
        _edge_body,
        out_type=(
            jax.ShapeDtypeStruct((NC, N, D), jnp.float32),
            jax.ShapeDtypeStruct((NC * N,), jnp.float32),
        ),
        mesh=mesh,
        compiler_params=pltpu.CompilerParams(needs_layout_passes=False,
                                             use_tc_tiling_on_sc=False),
        scratch_types=[
            [pltpu.VMEM((C,), jnp.int32)] * NB,
            [pltpu.VMEM((C,), jnp.int32)] * NB,
            [pltpu.VMEM((C, D), jnp.float32)] * NB,
            [pltpu.VMEM((C, D), jnp.float32)] * NB,
            [pltpu.VMEM((C, DE), jnp.float32)] * NB,
            [pltpu.VMEM((C, DE), jnp.float32)] * NB,
            [pltpu.VMEM((C,), jnp.float32)] * NB,
            pltpu.VMEM((256,), jnp.float32),
            pltpu.VMEM((800,), jnp.float32),
            pltpu.VMEM_SHARED((N, D), jnp.float32),
            pltpu.VMEM_SHARED((N,), jnp.float32),
            [pltpu.SemaphoreType.DMA] * NB,
            [pltpu.SemaphoreType.DMA] * NB,
        ],
    )
    return f(y, eigs2, src3, dst3)


def _combine_body(acc_ref, den_ref, out_ref):
    a = acc_ref[0] + acc_ref[1]
    d = den_ref[0, :, 0] + den_ref[1, :, 0]
    d = jnp.where(d == 0.0, 1.0, d)
    out_ref[...] = a / d[:, None]


def _combine(acc2, den2):
    den3 = den2.reshape(NC, N, 1)
    blk = N // 10
    return pl.pallas_call(
        _combine_body,
        out_shape=jax.ShapeDtypeStruct((N, D), jnp.float32),
        grid=(10,),
        in_specs=[
            pl.BlockSpec((NC, blk, D), lambda i: (0, i, 0)),
            pl.BlockSpec((NC, blk, 1), lambda i: (0, i, 0)),
        ],
        out_specs=pl.BlockSpec((blk, D), lambda i: (i, 0)),
    )(acc2, den3)


def kernel(x, indices, eigs, lambda0):
    y, eigs2 = _layernorm_and_scale(x, eigs, lambda0)
    src3 = indices[0].reshape(NW, NCHUNK, C)
    dst3 = indices[1].reshape(NW, NCHUNK, C)
    acc2, den2 = _edge_pass(y, eigs2, src3, dst3)
    return _combine(acc2, den2.reshape(NC, N))


# speedup vs baseline: 1.2044x; 1.2044x over previous
import jax
import jax.numpy as jnp
from jax import lax
from jax.experimental import pallas as pl
from jax.experimental.pallas import tpu as pltpu
from jax.experimental.pallas import tpu_sc as plsc

N = 10000
D = 128
DE = 16
E = 320000

NC = 2
NS = 16
NW = NC * NS
EW = E // NW
C = 80
NCHUNK = EW // C
NGRP = C // 16
NB = 2
ROWS_PER_TILE = N // NS
INV_SQRT_D = 1.0 / float(D) ** 0.5


def _ln_body(lam_ref, x_ref, e_ref, y_ref, e2_ref):
    x = x_ref[...]
    mean = jnp.mean(x, axis=-1, keepdims=True)
    var = jnp.mean((x - mean) ** 2, axis=-1, keepdims=True)
    y_ref[...] = (x - mean) / jnp.sqrt(var + 1e-5)
    scale = jnp.exp(0.5 * lam_ref[0])
    e2_ref[...] = e_ref[...] * scale


def _layernorm_and_scale(x, eigs, lambda0):
    return pl.pallas_call(
        _ln_body,
        out_shape=(
            jax.ShapeDtypeStruct((N, D), jnp.float32),
            jax.ShapeDtypeStruct((N, DE), jnp.float32),
        ),
        grid=(10,),
        in_specs=[
            pl.BlockSpec(memory_space=pltpu.SMEM),
            pl.BlockSpec((N // 10, D), lambda i: (i, 0)),
            pl.BlockSpec((N // 10, DE), lambda i: (i, 0)),
        ],
        out_specs=(
            pl.BlockSpec((N // 10, D), lambda i: (i, 0)),
            pl.BlockSpec((N // 10, DE), lambda i: (i, 0)),
        ),
    )(lambda0, x, eigs)


def _edge_body(y_hbm, eig_hbm, src_hbm, dst_hbm, acc_out, den_out,
               src_idx, dst_idx, ysrc, ydst, esrc, edst, valbuf, mbuf, zd,
               acc_sp, den_sp, sem_g, sem_s, sem_i):
    cid = lax.axis_index("c")
    sid = lax.axis_index("s")
    wid = cid * NS + sid

    def _zero_rows(r, _):
        for j in range(D // 16):
            ysrc[0][r, pl.ds(j * 16, 16)] = jnp.zeros((16,), jnp.float32)
        return 0

    lax.fori_loop(0, C, _zero_rows, 0)

    def _zero_zd(i, _):
        zd[pl.ds(i * 16, 16)] = jnp.zeros((16,), jnp.float32)
        return 0

    lax.fori_loop(0, 800 // 16, _zero_zd, 0)

    for k in range(7):
        pltpu.sync_copy(ysrc[0],
                        acc_sp.at[pl.ds(sid * ROWS_PER_TILE + k * C, C)])
    pltpu.sync_copy(ysrc[0].at[pl.ds(0, 65)],
                    acc_sp.at[pl.ds(sid * ROWS_PER_TILE + 7 * C, 65)])
    @pl.when(sid < 12)
    def _():
        pltpu.sync_copy(zd, den_sp.at[pl.ds(sid * 800, 800)])

    @pl.when(sid == 12)
    def _():
        pltpu.sync_copy(zd.at[pl.ds(0, 400)], den_sp.at[pl.ds(9600, 400)])

    plsc.subcore_barrier()

    def _issue_idx(ck, i):
        pltpu.async_copy(src_hbm.at[wid, ck], src_idx[i], sem_i[i])
        pltpu.async_copy(dst_hbm.at[wid, ck], dst_idx[i], sem_i[i])

    def _wait_idx(ck, i):
        pltpu.make_async_copy(src_hbm.at[wid, ck], src_idx[i],
                              sem_i[i]).wait()
        pltpu.make_async_copy(dst_hbm.at[wid, ck], dst_idx[i],
                              sem_i[i]).wait()

    def _issue_gather(b, i):
        pltpu.async_copy(y_hbm.at[src_idx[i]], ysrc[b], sem_g[b])
        pltpu.async_copy(y_hbm.at[dst_idx[i]], ydst[b], sem_g[b])
        pltpu.async_copy(eig_hbm.at[src_idx[i]], esrc[b], sem_g[b])
        pltpu.async_copy(eig_hbm.at[dst_idx[i]], edst[b], sem_g[b])

    def _wait_gather(b, i):
        pltpu.make_async_copy(y_hbm.at[src_idx[i]], ysrc[b], sem_g[b]).wait()
        pltpu.make_async_copy(y_hbm.at[dst_idx[i]], ydst[b], sem_g[b]).wait()
        pltpu.make_async_copy(eig_hbm.at[src_idx[i]], esrc[b],
                              sem_g[b]).wait()
        pltpu.make_async_copy(eig_hbm.at[dst_idx[i]], edst[b],
                              sem_g[b]).wait()

    def _issue_scatter(b, i):
        pltpu.async_copy(ysrc[b], acc_sp.at[src_idx[i]], sem_s[b], add=True)
        pltpu.async_copy(valbuf[b], den_sp.at[src_idx[i]], sem_s[b],
                         add=True)

    def _wait_scatter(b, i):
        pltpu.make_async_copy(ysrc[b], acc_sp.at[src_idx[i]],
                              sem_s[b]).wait()
        pltpu.make_async_copy(valbuf[b], den_sp.at[src_idx[i]],
                              sem_s[b]).wait()

    def _compute(b):
        lanes16 = lax.iota(jnp.int32, 16) * 16
        ridx = [lanes16 + l for l in range(16)]
        inv = jnp.full((16,), INV_SQRT_D, jnp.float32)

        def _group(g, _):
            for e in range(16):
                r = g * 16 + e
                qk = [ysrc[b][r, pl.ds(j * 16, 16)] for j in range(D // 16)]
                kk = [ydst[b][r, pl.ds(j * 16, 16)] for j in range(D // 16)]
                t0 = qk[0] * kk[0] + qk[1] * kk[1]
                t1 = qk[2] * kk[2] + qk[3] * kk[3]
                t2 = qk[4] * kk[4] + qk[5] * kk[5]
                t3 = qk[6] * kk[6] + qk[7] * kk[7]
                accy = (t0 + t1) + (t2 + t3)
                acc_t = accy * inv + esrc[b][r, :] * edst[b][r, :]
                mbuf[pl.ds(e * 16, 16)] = acc_t
            sv = plsc.load_gather(mbuf, [ridx[0]])
            for l in range(1, 16):
                sv = sv + plsc.load_gather(mbuf, [ridx[l]])
            vals = jnp.minimum(jnp.exp(sv), 5.0)
            valbuf[b][pl.ds(g * 16, 16)] = vals
            for e in range(16):
                r = g * 16 + e
                val = plsc.load_gather(valbuf[b], [jnp.full((16,), r,
                                                           jnp.int32)])
                for j in range(D // 16):
                    ysrc[b][r, pl.ds(j * 16, 16)] = (
                        ydst[b][r, pl.ds(j * 16, 16)] * val)
            return 0

        lax.fori_loop(0, NGRP, _group, 0)

    _issue_idx(0, 0)
    _issue_idx(1, 1)
    _wait_idx(0, 0)
    _issue_gather(0, 0)
    _wait_idx(1, 1)
    _issue_gather(1, 1)
    _issue_idx(2, 2)
    _issue_idx(3, 3)
    _wait_gather(0, 0)
    _compute(0)
    _issue_scatter(0, 0)

    def _steady(m, _):
        for bo in range(4):
            ck = 1 + 4 * m + bo
            b = (1 + bo) % NB
            nxt = (b + 1) % NB
            i = (1 + bo) % 4
            inxt = (2 + bo) % 4
            _wait_scatter(nxt, bo % 4)

            @pl.when(ck + 1 < NCHUNK)
            def _():
                _wait_idx(ck + 1, inxt)
                _issue_gather(nxt, inxt)

            @pl.when(ck + 3 < NCHUNK)
            def _():
                _issue_idx(ck + 3, bo % 4)

            _wait_gather(b, i)
            _compute(b)
            _issue_scatter(b, i)
        return 0

    lax.fori_loop(0, (NCHUNK - 1) // 4, _steady, 0)

    _wait_scatter(0, 0)

    plsc.subcore_barrier()

    @pl.when(sid < 15)
    def _():
        pltpu.sync_copy(acc_sp.at[pl.ds(sid * 632, 632)],
                        acc_out.at[cid, pl.ds(sid * 632, 632)])

    @pl.when(sid == 15)
    def _():
        pltpu.sync_copy(acc_sp.at[pl.ds(9480, 520)],
                        acc_out.at[cid, pl.ds(9480, 520)])

    @pl.when(sid < 12)
    def _():
        pltpu.sync_copy(den_sp.at[pl.ds(sid * 800, 800)],
                        den_out.at[pl.ds(cid * N + sid * 800, 800)])

    @pl.when(sid == 12)
    def _():
        pltpu.sync_copy(den_sp.at[pl.ds(9600, 400)],
                        den_out.at[pl.ds(cid * N + 9600, 400)])


def _edge_pass(y, eigs2, src3, dst3):
    mesh = plsc.VectorSubcoreMesh(core_axis_name="c", subcore_axis_name="s")
    f = pl.kernel(
        _edge_body,
        out_type=(
            jax.ShapeDtypeStruct((NC, N, D), jnp.float32),
            jax.ShapeDtypeStruct((NC * N,), jnp.float32),
        ),
        mesh=mesh,
        compiler_params=pltpu.CompilerParams(needs_layout_passes=False,
                                             use_tc_tiling_on_sc=False),
        scratch_types=[
            [pltpu.VMEM((C,), jnp.int32)] * 4,
            [pltpu.VMEM((C,), jnp.int32)] * 4,
            [pltpu.VMEM((C, D), jnp.float32)] * NB,
            [pltpu.VMEM((C, D), jnp.float32)] * NB,
            [pltpu.VMEM((C, DE), jnp.float32)] * NB,
            [pltpu.VMEM((C, DE), jnp.float32)] * NB,
            [pltpu.VMEM((C,), jnp.float32)] * NB,
            pltpu.VMEM((256,), jnp.float32),
            pltpu.VMEM((800,), jnp.float32),
            pltpu.VMEM_SHARED((N, D), jnp.float32),
            pltpu.VMEM_SHARED((N,), jnp.float32),
            [pltpu.SemaphoreType.DMA] * NB,
            [pltpu.SemaphoreType.DMA] * NB,
            [pltpu.SemaphoreType.DMA] * 4,
        ],
    )
    return f(y, eigs2, src3, dst3)


def _combine_body(acc_ref, den_ref, out_ref):
    a = acc_ref[0] + acc_ref[1]
    d = den_ref[0, :, 0] + den_ref[1, :, 0]
    d = jnp.where(d == 0.0, 1.0, d)
    out_ref[...] = a / d[:, None]


def _combine(acc2, den2):
    den3 = den2.reshape(NC, N, 1)
    blk = N // 10
    return pl.pallas_call(
        _combine_body,
        out_shape=jax.ShapeDtypeStruct((N, D), jnp.float32),
        grid=(10,),
        in_specs=[
            pl.BlockSpec((NC, blk, D), lambda i: (0, i, 0)),
            pl.BlockSpec((NC, blk, 1), lambda i: (0, i, 0)),
        ],
        out_specs=pl.BlockSpec((blk, D), lambda i: (i, 0)),
    )(acc2, den3)


def kernel(x, indices, eigs, lambda0):
    y, eigs2 = _layernorm_and_scale(x, eigs, lambda0)
    src3 = indices[0].reshape(NW, NCHUNK, C)
    dst3 = indices[1].reshape(NW, NCHUNK, C)
    acc2, den2 = _edge_pass(y, eigs2, src3, dst3)
    return _combine(acc2, den2.reshape(NC, N))


# speedup vs baseline: 1.2082x; 1.0031x over previous
import jax
import jax.numpy as jnp
from jax import lax
from jax.experimental import pallas as pl
from jax.experimental.pallas import tpu as pltpu
from jax.experimental.pallas import tpu_sc as plsc

N = 10000
D = 128
DE = 16
E = 320000

NC = 2
NS = 16
NW = NC * NS
EW = E // NW
C = 80
NCHUNK = EW // C
NGRP = C // 16
NB = 2
ROWS_PER_TILE = N // NS
INV_SQRT_D = 1.0 / float(D) ** 0.5


def _ln_body(lam_ref, x_ref, e_ref, y_ref, e2_ref):
    x = x_ref[...]
    mean = jnp.mean(x, axis=-1, keepdims=True)
    var = jnp.mean((x - mean) ** 2, axis=-1, keepdims=True)
    y_ref[...] = (x - mean) / jnp.sqrt(var + 1e-5)
    scale = jnp.exp(0.5 * lam_ref[0])
    e2_ref[...] = e_ref[...] * scale


def _layernorm_and_scale(x, eigs, lambda0):
    return pl.pallas_call(
        _ln_body,
        out_shape=(
            jax.ShapeDtypeStruct((N, D), jnp.float32),
            jax.ShapeDtypeStruct((N, DE), jnp.float32),
        ),
        grid=(10,),
        in_specs=[
            pl.BlockSpec(memory_space=pltpu.SMEM),
            pl.BlockSpec((N // 10, D), lambda i: (i, 0)),
            pl.BlockSpec((N // 10, DE), lambda i: (i, 0)),
        ],
        out_specs=(
            pl.BlockSpec((N // 10, D), lambda i: (i, 0)),
            pl.BlockSpec((N // 10, DE), lambda i: (i, 0)),
        ),
    )(lambda0, x, eigs)


def _edge_body(y_hbm, eig_hbm, src_hbm, dst_hbm, acc_out, den_out,
               src_idx, dst_idx, ysrc, ydst, esrc, edst, valbuf, mbuf, zd,
               acc_sp, den_sp, sem_g, sem_s, sem_i):
    cid = lax.axis_index("c")
    sid = lax.axis_index("s")
    wid = cid * NS + sid

    def _zero_rows(r, _):
        for j in range(D // 16):
            ysrc[0][r, pl.ds(j * 16, 16)] = jnp.zeros((16,), jnp.float32)
        return 0

    lax.fori_loop(0, C, _zero_rows, 0)

    def _zero_zd(i, _):
        zd[pl.ds(i * 16, 16)] = jnp.zeros((16,), jnp.float32)
        return 0

    lax.fori_loop(0, 800 // 16, _zero_zd, 0)

    for k in range(7):
        pltpu.sync_copy(ysrc[0],
                        acc_sp.at[pl.ds(sid * ROWS_PER_TILE + k * C, C)])
    pltpu.sync_copy(ysrc[0].at[pl.ds(0, 65)],
                    acc_sp.at[pl.ds(sid * ROWS_PER_TILE + 7 * C, 65)])
    @pl.when(sid < 12)
    def _():
        pltpu.sync_copy(zd, den_sp.at[pl.ds(sid * 800, 800)])

    @pl.when(sid == 12)
    def _():
        pltpu.sync_copy(zd.at[pl.ds(0, 400)], den_sp.at[pl.ds(9600, 400)])

    plsc.subcore_barrier()

    def _issue_idx(ck, i):
        pltpu.async_copy(src_hbm.at[wid, ck], src_idx[i], sem_i[i])
        pltpu.async_copy(dst_hbm.at[wid, ck], dst_idx[i], sem_i[i])

    def _wait_idx(ck, i):
        pltpu.make_async_copy(src_hbm.at[wid, ck], src_idx[i],
                              sem_i[i]).wait()
        pltpu.make_async_copy(dst_hbm.at[wid, ck], dst_idx[i],
                              sem_i[i]).wait()

    def _issue_gather(b, i):
        pltpu.async_copy(y_hbm.at[src_idx[i]], ysrc[b], sem_g[b])
        pltpu.async_copy(y_hbm.at[dst_idx[i]], ydst[b], sem_g[b])
        pltpu.async_copy(eig_hbm.at[src_idx[i]], esrc[b], sem_g[b])
        pltpu.async_copy(eig_hbm.at[dst_idx[i]], edst[b], sem_g[b])

    def _wait_gather(b, i):
        pltpu.make_async_copy(y_hbm.at[src_idx[i]], ysrc[b], sem_g[b]).wait()
        pltpu.make_async_copy(y_hbm.at[dst_idx[i]], ydst[b], sem_g[b]).wait()
        pltpu.make_async_copy(eig_hbm.at[src_idx[i]], esrc[b],
                              sem_g[b]).wait()
        pltpu.make_async_copy(eig_hbm.at[dst_idx[i]], edst[b],
                              sem_g[b]).wait()

    def _issue_scatter(b, i):
        pltpu.async_copy(ysrc[b], acc_sp.at[src_idx[i]], sem_s[b], add=True)
        pltpu.async_copy(valbuf[b], den_sp.at[src_idx[i]], sem_s[b],
                         add=True)

    def _wait_scatter(b, i):
        pltpu.make_async_copy(ysrc[b], acc_sp.at[src_idx[i]],
                              sem_s[b]).wait()
        pltpu.make_async_copy(valbuf[b], den_sp.at[src_idx[i]],
                              sem_s[b]).wait()

    def _compute(b):
        lanes16 = lax.iota(jnp.int32, 16) * 16
        ridx = [lanes16 + l for l in range(16)]
        inv = jnp.full((16,), INV_SQRT_D, jnp.float32)

        @plsc.parallel_loop(0, NGRP)
        def _group(g):
            for e in range(16):
                r = g * 16 + e
                qk = [ysrc[b][r, pl.ds(j * 16, 16)] for j in range(D // 16)]
                kk = [ydst[b][r, pl.ds(j * 16, 16)] for j in range(D // 16)]
                t0 = qk[0] * kk[0] + qk[1] * kk[1]
                t1 = qk[2] * kk[2] + qk[3] * kk[3]
                t2 = qk[4] * kk[4] + qk[5] * kk[5]
                t3 = qk[6] * kk[6] + qk[7] * kk[7]
                accy = (t0 + t1) + (t2 + t3)
                acc_t = accy * inv + esrc[b][r, :] * edst[b][r, :]
                mbuf[pl.ds(g * 256 + e * 16, 16)] = acc_t
            g256 = g * 256
            sv = plsc.load_gather(mbuf, [ridx[0] + g256])
            for l in range(1, 16):
                sv = sv + plsc.load_gather(mbuf, [ridx[l] + g256])
            vals = jnp.minimum(jnp.exp(sv), 5.0)
            valbuf[b][pl.ds(g * 16, 16)] = vals
            for e in range(16):
                r = g * 16 + e
                val = plsc.load_gather(valbuf[b], [jnp.full((16,), r,
                                                           jnp.int32)])
                for j in range(D // 16):
                    ysrc[b][r, pl.ds(j * 16, 16)] = (
                        ydst[b][r, pl.ds(j * 16, 16)] * val)

    _issue_idx(0, 0)
    _issue_idx(1, 1)
    _wait_idx(0, 0)
    _issue_gather(0, 0)
    _wait_idx(1, 1)
    _issue_gather(1, 1)
    _issue_idx(2, 2)
    _issue_idx(3, 3)
    _wait_gather(0, 0)
    _compute(0)
    _issue_scatter(0, 0)

    def _steady(m, _):
        for bo in range(4):
            ck = 1 + 4 * m + bo
            b = (1 + bo) % NB
            nxt = (b + 1) % NB
            i = (1 + bo) % 4
            inxt = (2 + bo) % 4
            _wait_scatter(nxt, bo % 4)

            @pl.when(ck + 1 < NCHUNK)
            def _():
                _wait_idx(ck + 1, inxt)
                _issue_gather(nxt, inxt)

            @pl.when(ck + 3 < NCHUNK)
            def _():
                _issue_idx(ck + 3, bo % 4)

            _wait_gather(b, i)
            _compute(b)
            _issue_scatter(b, i)
        return 0

    lax.fori_loop(0, (NCHUNK - 1) // 4, _steady, 0)

    _wait_scatter(0, 0)

    plsc.subcore_barrier()

    @pl.when(sid < 15)
    def _():
        pltpu.sync_copy(acc_sp.at[pl.ds(sid * 632, 632)],
                        acc_out.at[cid, pl.ds(sid * 632, 632)])

    @pl.when(sid == 15)
    def _():
        pltpu.sync_copy(acc_sp.at[pl.ds(9480, 520)],
                        acc_out.at[cid, pl.ds(9480, 520)])

    @pl.when(sid < 12)
    def _():
        pltpu.sync_copy(den_sp.at[pl.ds(sid * 800, 800)],
                        den_out.at[pl.ds(cid * N + sid * 800, 800)])

    @pl.when(sid == 12)
    def _():
        pltpu.sync_copy(den_sp.at[pl.ds(9600, 400)],
                        den_out.at[pl.ds(cid * N + 9600, 400)])


def _edge_pass(y, eigs2, src3, dst3):
    mesh = plsc.VectorSubcoreMesh(core_axis_name="c", subcore_axis_name="s")
    f = pl.kernel(
        _edge_body,
        out_type=(
            jax.ShapeDtypeStruct((NC, N, D), jnp.float32),
            jax.ShapeDtypeStruct((NC * N,), jnp.float32),
        ),
        mesh=mesh,
        compiler_params=pltpu.CompilerParams(needs_layout_passes=False,
                                             use_tc_tiling_on_sc=False),
        scratch_types=[
            [pltpu.VMEM((C,), jnp.int32)] * 4,
            [pltpu.VMEM((C,), jnp.int32)] * 4,
            [pltpu.VMEM((C, D), jnp.float32)] * NB,
            [pltpu.VMEM((C, D), jnp.float32)] * NB,
            [pltpu.VMEM((C, DE), jnp.float32)] * NB,
            [pltpu.VMEM((C, DE), jnp.float32)] * NB,
            [pltpu.VMEM((C,), jnp.float32)] * NB,
            pltpu.VMEM((NGRP * 256,), jnp.float32),
            pltpu.VMEM((800,), jnp.float32),
            pltpu.VMEM_SHARED((N, D), jnp.float32),
            pltpu.VMEM_SHARED((N,), jnp.float32),
            [pltpu.SemaphoreType.DMA] * NB,
            [pltpu.SemaphoreType.DMA] * NB,
            [pltpu.SemaphoreType.DMA] * 4,
        ],
    )
    return f(y, eigs2, src3, dst3)


def _combine_body(acc_ref, den_ref, out_ref):
    a = acc_ref[0] + acc_ref[1]
    d = den_ref[0, :, 0] + den_ref[1, :, 0]
    d = jnp.where(d == 0.0, 1.0, d)
    out_ref[...] = a / d[:, None]


def _combine(acc2, den2):
    den3 = den2.reshape(NC, N, 1)
    blk = N // 10
    return pl.pallas_call(
        _combine_body,
        out_shape=jax.ShapeDtypeStruct((N, D), jnp.float32),
        grid=(10,),
        in_specs=[
            pl.BlockSpec((NC, blk, D), lambda i: (0, i, 0)),
            pl.BlockSpec((NC, blk, 1), lambda i: (0, i, 0)),
        ],
        out_specs=pl.BlockSpec((blk, D), lambda i: (i, 0)),
    )(acc2, den3)


def kernel(x, indices, eigs, lambda0):
    y, eigs2 = _layernorm_and_scale(x, eigs, lambda0)
    src3 = indices[0].reshape(NW, NCHUNK, C)
    dst3 = indices[1].reshape(NW, NCHUNK, C)
    acc2, den2 = _edge_pass(y, eigs2, src3, dst3)
    return _combine(acc2, den2.reshape(NC, N))
